# baseline (speedup 1.0000x reference)
import functools

import jax
import jax.numpy as jnp
from jax import lax
from jax.experimental import pallas as pl
from jax.experimental.pallas import tpu as pltpu
from jax.experimental.pallas import tpu_sc as plsc

B = 8
N = 2048
K = 32
R = 1024

NC, NS, L = 2, 16, 16
NW = NC * NS
RW = (B * N) // NW
NPB = N // RW


def _topk_body(xb_ref, xr_ref, idx_ref, key_ref):
    _IMIN = jnp.int32(-2147483648)
    xb = xb_ref[0]
    xr = xr_ref[0]
    xx = jnp.sum(xb * xb, axis=0, keepdims=True)
    xxr = jnp.transpose(jnp.sum(xr * xr, axis=0, keepdims=True))
    dot = lax.dot_general(xr, xb, (((0,), (0,)), ((), ())),
                          preferred_element_type=jnp.float32)
    pd = 2.0 * dot - xxr - xx
    iota_n = lax.broadcasted_iota(jnp.int32, (1, N), 1)
    iota_r = lax.broadcasted_iota(jnp.int32, (R, 1), 0)
    iota_k = lax.broadcasted_iota(jnp.int32, (1, K), 1)
    pi = lax.bitcast_convert_type(pd, jnp.int32)
    mono = pi ^ (jnp.int32(0x7FFFFFFF) & (pi >> 31))
    key = (mono & jnp.int32(~2047)) | (jnp.int32(2047) - iota_n)
    row_g = iota_r + pl.program_id(1) * R
    key_ref[...] = jnp.where(iota_n == row_g, _IMIN, key)
    iacc0 = jnp.where(iota_k == 0, row_g, jnp.zeros((R, K), jnp.int32))

    def step(k, carry):
        iacc, mprev = carry
        kv = key_ref[...]
        m = jnp.max(jnp.where(kv < mprev, kv, _IMIN),
                    axis=1, keepdims=True)
        jstar = jnp.int32(2047) - (m & jnp.int32(2047))
        return jnp.where(iota_k == k, jstar, iacc), m

    carry = (iacc0, jnp.full((R, 1), jnp.int32(2147483647)))
    for k in range(1, K):
        carry = step(k, carry)
    iacc = carry[0]
    idx_ref[0] = jnp.transpose(iacc)


@jax.jit
def _tc_topk(x2):
    return pl.pallas_call(
        _topk_body,
        grid=(B, N // R),
        in_specs=[
            pl.BlockSpec((1, 3, N), lambda b, r: (b, 0, 0)),
            pl.BlockSpec((1, 3, R), lambda b, r: (b, 0, r)),
        ],
        out_specs=pl.BlockSpec((1, K, R), lambda b, r: (b, 0, r)),
        out_shape=jax.ShapeDtypeStruct((B, K, N), jnp.int32),
        scratch_shapes=[pltpu.VMEM((R, N), jnp.int32)],
    )(x2, x2)


def _sc_means_body(xf_hbm, idxt_hbm, out_hbm, xf_v, idx_v, acc_v):
    c = lax.axis_index("c")
    s = lax.axis_index("s")
    wid = s * NC + c
    b = wid // NPB
    base_n = (wid % NPB) * RW
    pltpu.sync_copy(xf_hbm.at[b], xf_v)
    pltpu.sync_copy(idxt_hbm.at[b, :, pl.ds(base_n, RW)], idx_v)

    def group(g, carry):
        col = g * L
        acc_a = [jnp.zeros((L,), jnp.float32) for _ in range(3)]
        s8 = [None, None, None]
        for j in range(K):
            idxv = idx_v[j, pl.ds(col, L)]
            for ch in range(3):
                acc_a[ch] = acc_a[ch] + plsc.load_gather(
                    xf_v, [idxv + (ch * N)])
            if j == 7:
                s8 = [a * 0.125 for a in acc_a]
        acc_z = [jnp.zeros((L,), jnp.float32) for _ in range(3)]
        for j in range(16):
            idxv = idx_v[j, pl.ds(col, L)]
            i3 = idxv * 3
            for ch in range(3):
                acc_z[ch] = acc_z[ch] + plsc.load_gather(xf_v, [i3 + ch])
        for ch in range(3):
            acc_v[ch, pl.ds(col, L)] = s8[ch]
            acc_v[3 + ch, pl.ds(col, L)] = acc_z[ch] * (1.0 / 16.0)
            acc_v[6 + ch, pl.ds(col, L)] = acc_a[ch] * (1.0 / 32.0)
        return carry

    lax.fori_loop(0, RW // L, group, 0)
    pltpu.sync_copy(acc_v, out_hbm.at[:, pl.ds(wid * RW, RW)])


@jax.jit
def _sc_means(xf, idxt):
    mesh = plsc.VectorSubcoreMesh(core_axis_name="c", subcore_axis_name="s")
    fn = functools.partial(
        pl.kernel,
        mesh=mesh,
        compiler_params=pltpu.CompilerParams(needs_layout_passes=False),
        out_type=jax.ShapeDtypeStruct((9, B * N), jnp.float32),
        scratch_types=[
            pltpu.VMEM((3 * N,), jnp.float32),
            pltpu.VMEM((K, RW), jnp.int32),
            pltpu.VMEM((9, RW), jnp.float32),
        ],
    )(_sc_means_body)
    return fn(xf, idxt)


def kernel(x):
    x2 = x.reshape(B, 3, N)
    idxt = _tc_topk(x2)
    out9 = _sc_means(x2.reshape(B, 3 * N), idxt)
    m8 = out9[0:3].reshape(3, B, N).transpose(1, 2, 0)
    m16 = out9[3:6].reshape(3, B, N).transpose(1, 2, 0)
    m32 = out9[6:9].reshape(3, B, N).transpose(1, 2, 0)
    concat_x = jnp.transpose(x2[:, :, :, None], (0, 2, 1, 3))
    for m in (m8, m16, m32):
        feature = m.reshape(B, N, 1, 1, 3)
        num_dims = concat_x.shape[3]
        concat_x = concat_x.reshape(B, N, 1, num_dims, 3)
        concat_x = jnp.concatenate([feature, concat_x], axis=3)
        concat_x = jnp.transpose(concat_x, (0, 4, 1, 3, 2))
    return jnp.transpose(concat_x, (0, 3, 1, 2, 4))


# speedup vs baseline: 1.6241x; 1.6241x over previous
import functools

import jax
import jax.numpy as jnp
from jax import lax
from jax.experimental import pallas as pl
from jax.experimental.pallas import tpu as pltpu
from jax.experimental.pallas import tpu_sc as plsc

B = 8
N = 2048
K = 32
R = 512

NC, NS, L = 2, 16, 16
NW = NC * NS
RW = (B * N) // NW
NPB = N // RW


def _topk_body(xb_ref, xr_ref, idx_ref, key_ref):
    _NEG = jnp.float32(-jnp.inf)
    xb = xb_ref[0]
    xr = xr_ref[0]
    xx = jnp.sum(xb * xb, axis=0, keepdims=True)
    xxr = jnp.transpose(jnp.sum(xr * xr, axis=0, keepdims=True))
    dot = lax.dot_general(xr, xb, (((0,), (0,)), ((), ())),
                          preferred_element_type=jnp.float32)
    pd = 2.0 * dot - xxr - xx
    iota_n = lax.broadcasted_iota(jnp.int32, (1, N), 1)
    iota_r = lax.broadcasted_iota(jnp.int32, (R, 1), 0)
    iota_k = lax.broadcasted_iota(jnp.int32, (1, K), 1)
    pi = lax.bitcast_convert_type(pd, jnp.int32)
    low = jnp.where(pd < 0.0, iota_n, jnp.int32(2047) - iota_n)
    keyf = lax.bitcast_convert_type((pi & jnp.int32(~2047)) | low,
                                    jnp.float32)
    row_g = iota_r + pl.program_id(1) * R
    key_ref[...] = jnp.where(iota_n == row_g, _NEG, keyf)
    iacc0 = jnp.where(iota_k == 0, row_g, jnp.zeros((R, K), jnp.int32))

    def step(k, carry):
        iacc, mprev = carry
        kv = key_ref[...]
        m = jnp.max(jnp.where(kv < mprev, kv, _NEG),
                    axis=1, keepdims=True)
        mb = lax.bitcast_convert_type(m, jnp.int32) & jnp.int32(2047)
        jstar = jnp.where(m < 0.0, mb, jnp.int32(2047) - mb)
        return jnp.where(iota_k == k, jstar, iacc), m

    carry = (iacc0, jnp.full((R, 1), jnp.float32(jnp.inf)))
    for k in range(1, K):
        carry = step(k, carry)
    iacc = carry[0]
    idx_ref[0] = jnp.transpose(iacc)


@jax.jit
def _tc_topk(x2):
    return pl.pallas_call(
        _topk_body,
        grid=(B, N // R),
        in_specs=[
            pl.BlockSpec((1, 3, N), lambda b, r: (b, 0, 0)),
            pl.BlockSpec((1, 3, R), lambda b, r: (b, 0, r)),
        ],
        out_specs=pl.BlockSpec((1, K, R), lambda b, r: (b, 0, r)),
        out_shape=jax.ShapeDtypeStruct((B, K, N), jnp.int32),
        scratch_shapes=[pltpu.VMEM((R, N), jnp.float32)],
    )(x2, x2)


def _sc_means_body(xf_hbm, idxt_hbm, out_hbm, xf_v, idx_v, acc_v):
    c = lax.axis_index("c")
    s = lax.axis_index("s")
    wid = s * NC + c
    b = wid // NPB
    base_n = (wid % NPB) * RW
    pltpu.sync_copy(xf_hbm.at[b], xf_v)
    pltpu.sync_copy(idxt_hbm.at[b, :, pl.ds(base_n, RW)], idx_v)

    def group(g, carry):
        col = g * L
        acc_a = [jnp.zeros((L,), jnp.float32) for _ in range(3)]
        s8 = [None, None, None]
        for j in range(K):
            idxv = idx_v[j, pl.ds(col, L)]
            for ch in range(3):
                acc_a[ch] = acc_a[ch] + plsc.load_gather(
                    xf_v, [idxv + (ch * N)])
            if j == 7:
                s8 = [a * 0.125 for a in acc_a]
        acc_z = [jnp.zeros((L,), jnp.float32) for _ in range(3)]
        for j in range(16):
            idxv = idx_v[j, pl.ds(col, L)]
            i3 = idxv * 3
            for ch in range(3):
                acc_z[ch] = acc_z[ch] + plsc.load_gather(xf_v, [i3 + ch])
        for ch in range(3):
            acc_v[ch, pl.ds(col, L)] = s8[ch]
            acc_v[3 + ch, pl.ds(col, L)] = acc_z[ch] * (1.0 / 16.0)
            acc_v[6 + ch, pl.ds(col, L)] = acc_a[ch] * (1.0 / 32.0)
        return carry

    lax.fori_loop(0, RW // L, group, 0)
    pltpu.sync_copy(acc_v, out_hbm.at[:, pl.ds(wid * RW, RW)])


@jax.jit
def _sc_means(xf, idxt):
    mesh = plsc.VectorSubcoreMesh(core_axis_name="c", subcore_axis_name="s")
    fn = functools.partial(
        pl.kernel,
        mesh=mesh,
        compiler_params=pltpu.CompilerParams(needs_layout_passes=False),
        out_type=jax.ShapeDtypeStruct((9, B * N), jnp.float32),
        scratch_types=[
            pltpu.VMEM((3 * N,), jnp.float32),
            pltpu.VMEM((K, RW), jnp.int32),
            pltpu.VMEM((9, RW), jnp.float32),
        ],
    )(_sc_means_body)
    return fn(xf, idxt)


def kernel(x):
    x2 = x.reshape(B, 3, N)
    idxt = _tc_topk(x2)
    out9 = _sc_means(x2.reshape(B, 3 * N), idxt)
    m8 = out9[0:3].reshape(3, B, N).transpose(1, 2, 0)
    m16 = out9[3:6].reshape(3, B, N).transpose(1, 2, 0)
    m32 = out9[6:9].reshape(3, B, N).transpose(1, 2, 0)
    concat_x = jnp.transpose(x2[:, :, :, None], (0, 2, 1, 3))
    for m in (m8, m16, m32):
        feature = m.reshape(B, N, 1, 1, 3)
        num_dims = concat_x.shape[3]
        concat_x = concat_x.reshape(B, N, 1, num_dims, 3)
        concat_x = jnp.concatenate([feature, concat_x], axis=3)
        concat_x = jnp.transpose(concat_x, (0, 4, 1, 3, 2))
    return jnp.transpose(concat_x, (0, 3, 1, 2, 4))


# speedup vs baseline: 1.7194x; 1.0587x over previous
import functools

import jax
import jax.numpy as jnp
from jax import lax
from jax.experimental import pallas as pl
from jax.experimental.pallas import tpu as pltpu
from jax.experimental.pallas import tpu_sc as plsc

B = 8
N = 2048
K = 32
R = 512

NC, NS, L = 2, 16, 16
NW = NC * NS
RW = (B * N) // NW
NPB = N // RW


def _topk_body(xb_ref, xr_ref, idx_ref, key_ref):
    _NEG = jnp.float32(-jnp.inf)
    xb = xb_ref[0]
    xr = xr_ref[0]
    xx = jnp.sum(xb * xb, axis=0, keepdims=True)
    xxr = jnp.transpose(jnp.sum(xr * xr, axis=0, keepdims=True))
    dot = lax.dot_general(xr, xb, (((0,), (0,)), ((), ())),
                          preferred_element_type=jnp.float32)
    pd = 2.0 * dot - xxr - xx
    iota_n = lax.broadcasted_iota(jnp.int32, (1, N), 1)
    iota_r = lax.broadcasted_iota(jnp.int32, (R, 1), 0)
    iota_k = lax.broadcasted_iota(jnp.int32, (1, K), 1)
    pi = lax.bitcast_convert_type(pd, jnp.int32)
    low = jnp.where(pd < 0.0, iota_n, jnp.int32(2047) - iota_n)
    keyf = lax.bitcast_convert_type((pi & jnp.int32(~2047)) | low,
                                    jnp.float32)
    row_g = iota_r + pl.program_id(1) * R
    key_ref[...] = jnp.where(iota_n == row_g, _NEG, keyf)

    def step(k, carry):
        macc, mprev = carry
        kv = key_ref[...]
        m = jnp.max(jnp.where(kv < mprev, kv, _NEG),
                    axis=1, keepdims=True)
        return jnp.where(iota_k == k, m, macc), m

    carry = (jnp.zeros((R, K), jnp.float32),
             jnp.full((R, 1), jnp.float32(jnp.inf)))
    for k in range(1, K):
        carry = step(k, carry)
    macc = carry[0]
    mb = lax.bitcast_convert_type(macc, jnp.int32) & jnp.int32(2047)
    iacc = jnp.where(macc < 0.0, mb, jnp.int32(2047) - mb)
    iacc = jnp.where(iota_k == 0, row_g, iacc)
    idx_ref[0] = jnp.transpose(iacc)


@jax.jit
def _tc_topk(x2):
    return pl.pallas_call(
        _topk_body,
        grid=(B, N // R),
        in_specs=[
            pl.BlockSpec((1, 3, N), lambda b, r: (b, 0, 0)),
            pl.BlockSpec((1, 3, R), lambda b, r: (b, 0, r)),
        ],
        out_specs=pl.BlockSpec((1, K, R), lambda b, r: (b, 0, r)),
        out_shape=jax.ShapeDtypeStruct((B, K, N), jnp.int32),
        scratch_shapes=[pltpu.VMEM((R, N), jnp.float32)],
    )(x2, x2)


def _sc_means_body(xf_hbm, idxt_hbm, out_hbm, xf_v, idx_v, acc_v):
    c = lax.axis_index("c")
    s = lax.axis_index("s")
    wid = s * NC + c
    b = wid // NPB
    base_n = (wid % NPB) * RW
    pltpu.sync_copy(xf_hbm.at[b], xf_v)
    pltpu.sync_copy(idxt_hbm.at[b, :, pl.ds(base_n, RW)], idx_v)

    def group(g, carry):
        col = g * L
        acc_a = [jnp.zeros((L,), jnp.float32) for _ in range(3)]
        s8 = [None, None, None]
        for j in range(K):
            idxv = idx_v[j, pl.ds(col, L)]
            for ch in range(3):
                acc_a[ch] = acc_a[ch] + plsc.load_gather(
                    xf_v, [idxv + (ch * N)])
            if j == 7:
                s8 = [a * 0.125 for a in acc_a]
        acc_z = [jnp.zeros((L,), jnp.float32) for _ in range(3)]
        for j in range(16):
            idxv = idx_v[j, pl.ds(col, L)]
            i3 = idxv * 3
            for ch in range(3):
                acc_z[ch] = acc_z[ch] + plsc.load_gather(xf_v, [i3 + ch])
        for ch in range(3):
            acc_v[ch, pl.ds(col, L)] = s8[ch]
            acc_v[3 + ch, pl.ds(col, L)] = acc_z[ch] * (1.0 / 16.0)
            acc_v[6 + ch, pl.ds(col, L)] = acc_a[ch] * (1.0 / 32.0)
        return carry

    lax.fori_loop(0, RW // L, group, 0)
    pltpu.sync_copy(acc_v, out_hbm.at[:, pl.ds(wid * RW, RW)])


@jax.jit
def _sc_means(xf, idxt):
    mesh = plsc.VectorSubcoreMesh(core_axis_name="c", subcore_axis_name="s")
    fn = functools.partial(
        pl.kernel,
        mesh=mesh,
        compiler_params=pltpu.CompilerParams(needs_layout_passes=False),
        out_type=jax.ShapeDtypeStruct((9, B * N), jnp.float32),
        scratch_types=[
            pltpu.VMEM((3 * N,), jnp.float32),
            pltpu.VMEM((K, RW), jnp.int32),
            pltpu.VMEM((9, RW), jnp.float32),
        ],
    )(_sc_means_body)
    return fn(xf, idxt)


def kernel(x):
    x2 = x.reshape(B, 3, N)
    idxt = _tc_topk(x2)
    out9 = _sc_means(x2.reshape(B, 3 * N), idxt)
    m8 = out9[0:3].reshape(3, B, N).transpose(1, 2, 0)
    m16 = out9[3:6].reshape(3, B, N).transpose(1, 2, 0)
    m32 = out9[6:9].reshape(3, B, N).transpose(1, 2, 0)
    concat_x = jnp.transpose(x2[:, :, :, None], (0, 2, 1, 3))
    for m in (m8, m16, m32):
        feature = m.reshape(B, N, 1, 1, 3)
        num_dims = concat_x.shape[3]
        concat_x = concat_x.reshape(B, N, 1, num_dims, 3)
        concat_x = jnp.concatenate([feature, concat_x], axis=3)
        concat_x = jnp.transpose(concat_x, (0, 4, 1, 3, 2))
    return jnp.transpose(concat_x, (0, 3, 1, 2, 4))


# speedup vs baseline: 1.8732x; 1.0894x over previous
import functools

import jax
import jax.numpy as jnp
from jax import lax
from jax.experimental import pallas as pl
from jax.experimental.pallas import tpu as pltpu
from jax.experimental.pallas import tpu_sc as plsc

B = 8
N = 2048
K = 32
R = 512

NC, NS, L = 2, 16, 16
NW = NC * NS
RW = (B * N) // NW
NPB = N // RW


def _topk_body(xb_ref, xr_ref, idx_ref, key_ref):
    _NEG = jnp.float32(-jnp.inf)
    xb = xb_ref[0]
    xr = xr_ref[0]
    xx = jnp.sum(xb * xb, axis=0, keepdims=True)
    xxr = jnp.transpose(jnp.sum(xr * xr, axis=0, keepdims=True))
    dot = lax.dot_general(xr, xb, (((0,), (0,)), ((), ())),
                          preferred_element_type=jnp.float32)
    pd = 2.0 * dot - xxr - xx
    iota_n = lax.broadcasted_iota(jnp.int32, (1, N), 1)
    iota_r = lax.broadcasted_iota(jnp.int32, (R, 1), 0)
    iota_k = lax.broadcasted_iota(jnp.int32, (1, K), 1)
    pi = lax.bitcast_convert_type(pd, jnp.int32)
    low = jnp.where(pd < 0.0, iota_n, jnp.int32(2047) - iota_n)
    keyf = lax.bitcast_convert_type((pi & jnp.int32(~2047)) | low,
                                    jnp.float32)
    row_g = iota_r + pl.program_id(1) * R
    key_ref[...] = jnp.where(iota_n == row_g, _NEG, keyf)

    def step(k, carry):
        macc, mprev = carry
        kv = key_ref[...]
        m = jnp.max(jnp.where(kv < mprev, kv, _NEG),
                    axis=1, keepdims=True)
        return jnp.where(iota_k == k, m, macc), m

    carry = (jnp.zeros((R, K), jnp.float32),
             jnp.full((R, 1), jnp.float32(jnp.inf)))
    for k in range(1, K):
        carry = step(k, carry)
    macc = carry[0]
    mb = lax.bitcast_convert_type(macc, jnp.int32) & jnp.int32(2047)
    iacc = jnp.where(macc < 0.0, mb, jnp.int32(2047) - mb)
    iacc = jnp.where(iota_k == 0, row_g, iacc)
    idx_ref[0] = jnp.transpose(iacc)


@jax.jit
def _tc_topk(x2):
    return pl.pallas_call(
        _topk_body,
        grid=(B, N // R),
        in_specs=[
            pl.BlockSpec((1, 3, N), lambda b, r: (b, 0, 0)),
            pl.BlockSpec((1, 3, R), lambda b, r: (b, 0, r)),
        ],
        out_specs=pl.BlockSpec((1, K, R), lambda b, r: (b, 0, r)),
        out_shape=jax.ShapeDtypeStruct((B, K, N), jnp.int32),
        scratch_shapes=[pltpu.VMEM((R, N), jnp.float32)],
    )(x2, x2)


def _dest_table():
    import numpy as np
    col = np.arange(B * N, dtype=np.int64).reshape(B, N)
    def mids(r0):
        return np.stack([(r0 + c) * B * N + col for c in range(3)], axis=-1)
    m8_i, m16_i, m32_i = mids(0), mids(3), mids(6)
    x2_i = np.stack([(9 + c) * B * N + col for c in range(3)], axis=1)
    concat_x = np.transpose(x2_i[:, :, :, None], (0, 2, 1, 3))
    for m in (m8_i, m16_i, m32_i):
        feature = m.reshape(B, N, 1, 1, 3)
        nd = concat_x.shape[3]
        concat_x = concat_x.reshape(B, N, 1, nd, 3)
        concat_x = np.concatenate([feature, concat_x], axis=3)
        concat_x = np.transpose(concat_x, (0, 4, 1, 3, 2))
    out_ids = np.transpose(concat_x, (0, 3, 1, 2, 4)).reshape(-1)
    return out_ids.astype(np.int32)


_OUT_IDS = _dest_table()


def _sc_means_body(xf_hbm, idxt_hbm, out_hbm, xf_v, idx_v, acc_v):
    c = lax.axis_index("c")
    s = lax.axis_index("s")
    wid = s * NC + c
    b = wid // NPB
    base_n = (wid % NPB) * RW
    pltpu.sync_copy(xf_hbm.at[b], xf_v)
    pltpu.sync_copy(idxt_hbm.at[b, :, pl.ds(base_n, RW)], idx_v)

    def group(g, carry):
        col = g * L
        acc_a = [jnp.zeros((L,), jnp.float32) for _ in range(3)]
        s8 = [None, None, None]
        for j in range(K):
            idxv = idx_v[j, pl.ds(col, L)]
            for ch in range(3):
                acc_a[ch] = acc_a[ch] + plsc.load_gather(
                    xf_v, [idxv + (ch * N)])
            if j == 7:
                s8 = [a * 0.125 for a in acc_a]
        acc_z = [jnp.zeros((L,), jnp.float32) for _ in range(3)]
        for j in range(16):
            idxv = idx_v[j, pl.ds(col, L)]
            i3 = idxv * 3
            for ch in range(3):
                acc_z[ch] = acc_z[ch] + plsc.load_gather(xf_v, [i3 + ch])
        for ch in range(3):
            acc_v[ch, pl.ds(col, L)] = s8[ch]
            acc_v[3 + ch, pl.ds(col, L)] = acc_z[ch] * (1.0 / 16.0)
            acc_v[6 + ch, pl.ds(col, L)] = acc_a[ch] * (1.0 / 32.0)
            acc_v[9 + ch, pl.ds(col, L)] = xf_v[pl.ds(ch * N + base_n + col, L)]
        return carry

    lax.fori_loop(0, RW // L, group, 0)
    pltpu.sync_copy(acc_v, out_hbm.at[:, pl.ds(wid * RW, RW)])


@jax.jit
def _sc_means(xf, idxt):
    mesh = plsc.VectorSubcoreMesh(core_axis_name="c", subcore_axis_name="s")
    fn = functools.partial(
        pl.kernel,
        mesh=mesh,
        compiler_params=pltpu.CompilerParams(needs_layout_passes=False),
        out_type=jax.ShapeDtypeStruct((12, B * N), jnp.float32),
        scratch_types=[
            pltpu.VMEM((3 * N,), jnp.float32),
            pltpu.VMEM((K, RW), jnp.int32),
            pltpu.VMEM((12, RW), jnp.float32),
        ],
    )(_sc_means_body)
    return fn(xf, idxt)


def kernel(x):
    x2 = x.reshape(B, 3, N)
    idxt = _tc_topk(x2)
    v = _sc_means(x2.reshape(B, 3 * N), idxt)
    flat = jnp.take(v.reshape(-1), jnp.asarray(_OUT_IDS))
    return flat.reshape(B, 4, 3, N, 1)


# speedup vs baseline: 1.9082x; 1.0187x over previous
import functools

import jax
import jax.numpy as jnp
from jax import lax
from jax.experimental import pallas as pl
from jax.experimental.pallas import tpu as pltpu
from jax.experimental.pallas import tpu_sc as plsc

B = 8
N = 2048
K = 32
R = 512

NC, NS, L = 2, 16, 16
NW = NC * NS
RW = (B * N) // NW
NPB = N // RW


def _topk_body(xb_ref, xr_ref, idx_ref, key_ref):
    _NEG = jnp.float32(-jnp.inf)
    xb = xb_ref[0]
    xr = xr_ref[0]
    xx = jnp.sum(xb * xb, axis=0, keepdims=True)
    xxr = jnp.transpose(jnp.sum(xr * xr, axis=0, keepdims=True))
    dot = lax.dot_general(xr, xb, (((0,), (0,)), ((), ())),
                          preferred_element_type=jnp.float32)
    pd = 2.0 * dot - xxr - xx
    iota_n = lax.broadcasted_iota(jnp.int32, (1, N), 1)
    iota_r = lax.broadcasted_iota(jnp.int32, (R, 1), 0)
    iota_k = lax.broadcasted_iota(jnp.int32, (1, K), 1)
    pi = lax.bitcast_convert_type(pd, jnp.int32)
    low = jnp.where(pd < 0.0, iota_n, jnp.int32(2047) - iota_n)
    keyf = lax.bitcast_convert_type((pi & jnp.int32(~2047)) | low,
                                    jnp.float32)
    row_g = iota_r + pl.program_id(1) * R
    key_ref[...] = jnp.where(iota_n == row_g, _NEG, keyf)

    def step(k, carry):
        macc, mprev = carry
        kv = key_ref[...]
        m = jnp.max(jnp.where(kv < mprev, kv, _NEG),
                    axis=1, keepdims=True)
        return jnp.where(iota_k == k, m, macc), m

    m1 = jnp.max(key_ref[...], axis=1, keepdims=True)
    macc = jnp.where(iota_k == 1, m1, jnp.zeros((R, K), jnp.float32))
    carry = (macc, m1)
    for k in range(2, K):
        carry = step(k, carry)
    macc = carry[0]
    mb = lax.bitcast_convert_type(macc, jnp.int32) & jnp.int32(2047)
    iacc = jnp.where(macc < 0.0, mb, jnp.int32(2047) - mb)
    iacc = jnp.where(iota_k == 0, row_g, iacc)
    idx_ref[0] = jnp.transpose(iacc)


@jax.jit
def _tc_topk(x2):
    return pl.pallas_call(
        _topk_body,
        grid=(B, N // R),
        in_specs=[
            pl.BlockSpec((1, 3, N), lambda b, r: (b, 0, 0)),
            pl.BlockSpec((1, 3, R), lambda b, r: (b, 0, r)),
        ],
        out_specs=pl.BlockSpec((1, K, R), lambda b, r: (b, 0, r)),
        out_shape=jax.ShapeDtypeStruct((B, K, N), jnp.int32),
        scratch_shapes=[pltpu.VMEM((R, N), jnp.float32)],
    )(x2, x2)


def _dest_table():
    import numpy as np
    col = np.arange(B * N, dtype=np.int64).reshape(B, N)
    def mids(r0):
        return np.stack([(r0 + c) * B * N + col for c in range(3)], axis=-1)
    m8_i, m16_i, m32_i = mids(0), mids(3), mids(6)
    x2_i = np.stack([(9 + c) * B * N + col for c in range(3)], axis=1)
    concat_x = np.transpose(x2_i[:, :, :, None], (0, 2, 1, 3))
    for m in (m8_i, m16_i, m32_i):
        feature = m.reshape(B, N, 1, 1, 3)
        nd = concat_x.shape[3]
        concat_x = concat_x.reshape(B, N, 1, nd, 3)
        concat_x = np.concatenate([feature, concat_x], axis=3)
        concat_x = np.transpose(concat_x, (0, 4, 1, 3, 2))
    out_ids = np.transpose(concat_x, (0, 3, 1, 2, 4)).reshape(-1)
    return out_ids.astype(np.int32)


_OUT_IDS = _dest_table()


def _sc_means_body(xf_hbm, idxt_hbm, out_hbm, xf_v, idx_v, acc_v):
    c = lax.axis_index("c")
    s = lax.axis_index("s")
    wid = s * NC + c
    b = wid // NPB
    base_n = (wid % NPB) * RW
    pltpu.sync_copy(xf_hbm.at[b], xf_v)
    pltpu.sync_copy(idxt_hbm.at[b, :, pl.ds(base_n, RW)], idx_v)

    def group(g, carry):
        col = g * L
        acc_a = [jnp.zeros((L,), jnp.float32) for _ in range(3)]
        s8 = [None, None, None]
        for j in range(K):
            idxv = idx_v[j, pl.ds(col, L)]
            for ch in range(3):
                acc_a[ch] = acc_a[ch] + plsc.load_gather(
                    xf_v, [idxv + (ch * N)])
            if j == 7:
                s8 = [a * 0.125 for a in acc_a]
        acc_z = [jnp.zeros((L,), jnp.float32) for _ in range(3)]
        for j in range(16):
            idxv = idx_v[j, pl.ds(col, L)]
            i3 = idxv * 3
            for ch in range(3):
                acc_z[ch] = acc_z[ch] + plsc.load_gather(xf_v, [i3 + ch])
        for ch in range(3):
            acc_v[ch, pl.ds(col, L)] = s8[ch]
            acc_v[3 + ch, pl.ds(col, L)] = acc_z[ch] * (1.0 / 16.0)
            acc_v[6 + ch, pl.ds(col, L)] = acc_a[ch] * (1.0 / 32.0)
            acc_v[9 + ch, pl.ds(col, L)] = xf_v[pl.ds(ch * N + base_n + col, L)]
        return carry

    lax.fori_loop(0, RW // L, group, 0)
    pltpu.sync_copy(acc_v, out_hbm.at[:, pl.ds(wid * RW, RW)])


@jax.jit
def _sc_means(xf, idxt):
    mesh = plsc.VectorSubcoreMesh(core_axis_name="c", subcore_axis_name="s")
    fn = functools.partial(
        pl.kernel,
        mesh=mesh,
        compiler_params=pltpu.CompilerParams(needs_layout_passes=False),
        out_type=jax.ShapeDtypeStruct((12, B * N), jnp.float32),
        scratch_types=[
            pltpu.VMEM((3 * N,), jnp.float32),
            pltpu.VMEM((K, RW), jnp.int32),
            pltpu.VMEM((12, RW), jnp.float32),
        ],
    )(_sc_means_body)
    return fn(xf, idxt)


def kernel(x):
    x2 = x.reshape(B, 3, N)
    idxt = _tc_topk(x2)
    v = _sc_means(x2.reshape(B, 3 * N), idxt)
    flat = jnp.take(v.reshape(-1), jnp.asarray(_OUT_IDS))
    return flat.reshape(B, 4, 3, N, 1)
